# baseline (speedup 1.0000x reference)
import functools

import jax
import jax.numpy as jnp
from jax import lax
from jax.experimental import pallas as pl
from jax.experimental.pallas import tpu as pltpu
from jax.experimental.pallas import tpu_sc as plsc

_N = 10000
_D = 128
_G = 64

_NC = 2
_NS = 16
_NW = _NC * _NS
_CHUNK = 128

_HALF = _N // 2
_HROWS = 5120
_HSTRIPE = _HROWS // _NS

_N_CNT = 10240


def _sc_mesh():
    return plsc.VectorSubcoreMesh(core_axis_name="c", subcore_axis_name="s")


def _make_prep_kernel(e_pad):
    ew = e_pad // _NW
    copies = 8
    per_copy = ew // copies
    assert per_copy * copies == ew and per_copy % 16 == 0 and per_copy % 8 == 0

    @functools.partial(
        pl.kernel,
        out_type=(
            jax.ShapeDtypeStruct((_NW, _N_CNT), jnp.float32),
            jax.ShapeDtypeStruct((2, e_pad), jnp.int32),
            jax.ShapeDtypeStruct((2, e_pad), jnp.int32),
        ),
        mesh=_sc_mesh(),
        scratch_types=[
            pltpu.VMEM((_N_CNT,), jnp.float32),
            pltpu.VMEM((per_copy,), jnp.int32),
            pltpu.VMEM((per_copy,), jnp.int32),
            pltpu.VMEM((2, per_copy), jnp.int32),
            pltpu.VMEM((2, per_copy), jnp.int32),
        ],
        compiler_params=pltpu.CompilerParams(needs_layout_passes=False),
    )
    def prep_kernel(src_hbm, dst_hbm, cnt_hbm, srcl_hbm, dstl_hbm,
                    cnt_v, sbuf, dbuf, slocal, dlocal):
        c = lax.axis_index("c")
        s = lax.axis_index("s")
        wid = s * _NC + c
        zero16 = jnp.zeros((16,), jnp.float32)
        ones16 = jnp.ones((16,), jnp.float32)
        half16 = jnp.full((16,), _HALF, jnp.int32)

        def z_body(i, carry):
            cnt_v[pl.ds(i * 16, 16)] = zero16
            return carry
        lax.fori_loop(0, _N_CNT // 16, z_body, 0)

        def outer(j, carry):
            base = wid * ew + j * per_copy
            pltpu.sync_copy(src_hbm.at[pl.ds(base, per_copy)], sbuf)
            pltpu.sync_copy(dst_hbm.at[pl.ds(base, per_copy)], dbuf)

            def inner(k, c2):
                sl = pl.ds(k * 16, 16)
                sv = sbuf[sl]
                dv = dbuf[sl]
                plsc.addupdate_scatter(cnt_v, (dv,), ones16)
                s_hi = sv - half16
                d_hi = dv - half16
                slocal[0, sl] = jnp.where(sv < half16, sv, half16)
                slocal[1, sl] = jnp.where(s_hi >= 0, s_hi, half16)
                dlocal[0, sl] = jnp.where(dv < half16, dv, half16)
                dlocal[1, sl] = jnp.where(d_hi >= 0, d_hi, half16)
                return c2
            lax.fori_loop(0, per_copy // 16, inner, 0)
            for h in (0, 1):
                pltpu.sync_copy(slocal.at[h],
                                srcl_hbm.at[h].at[pl.ds(base, per_copy)])
                pltpu.sync_copy(dlocal.at[h],
                                dstl_hbm.at[h].at[pl.ds(base, per_copy)])
            return carry
        lax.fori_loop(0, copies, outer, 0)

        pltpu.sync_copy(cnt_v, cnt_hbm.at[wid])

    return prep_kernel


def _make_scatter_kernel(e_pad):
    nchunk = e_pad // (_NS * _CHUNK)
    assert nchunk * _NS * _CHUNK == e_pad and nchunk % 2 == 0

    @functools.partial(
        pl.kernel,
        out_type=jax.ShapeDtypeStruct((_NC, 2, _HROWS, _D), jnp.float32),
        mesh=_sc_mesh(),
        scratch_types=[
            pltpu.VMEM((2, _CHUNK), jnp.int32),
            pltpu.VMEM((2, _CHUNK), jnp.int32),
            pltpu.VMEM((_CHUNK, _D), jnp.float32),
            pltpu.VMEM((_CHUNK, _D), jnp.float32),
            pltpu.VMEM_SHARED((_HROWS, _D), jnp.float32),
            pltpu.VMEM_SHARED((_HROWS, _D), jnp.float32),
            pltpu.SemaphoreType.DMA,
            pltpu.SemaphoreType.DMA,
        ],
    )
    def scatter_kernel(hs_hbm, srcl_hbm, dstl_hbm, zeros_hbm, out_hbm,
                       sbufs, dbufs, rows0, rows1, hs_sh, acc, sem0, sem1):
        q = lax.axis_index("c")
        s = lax.axis_index("s")
        rows = (rows0, rows1)
        sems = (sem0, sem1)
        stripe = pl.ds(s * _HSTRIPE, _HSTRIPE)

        pltpu.sync_copy(hs_hbm.at[q].at[stripe], hs_sh.at[stripe])

        for b in (0, 1):
            hb = lax.bitwise_xor(q, b)
            pltpu.sync_copy(zeros_hbm.at[stripe], acc.at[stripe])
            plsc.subcore_barrier()

            def issue(ci, k):
                row = s * nchunk + ci
                pltpu.sync_copy(srcl_hbm.at[q].at[row], sbufs.at[k])
                pltpu.sync_copy(dstl_hbm.at[hb].at[row], dbufs.at[k])
                pltpu.async_copy(hs_sh.at[sbufs.at[k]], rows[k], sems[k])

            def wait_scatter(k):
                pltpu.make_async_copy(hs_sh.at[sbufs.at[k]], rows[k],
                                      sems[k]).wait()
                pltpu.sync_copy(rows[k], acc.at[dbufs.at[k]], add=True)

            issue(0, 0)

            def step(g, carry):
                for k in (0, 1):
                    ci = 2 * g + k

                    @pl.when(ci + 1 < nchunk)
                    def _():
                        issue(ci + 1, 1 - k)
                    wait_scatter(k)
                return carry
            lax.fori_loop(0, nchunk // 2, step, 0)
            plsc.subcore_barrier()
            pltpu.sync_copy(acc.at[stripe], out_hbm.at[q].at[b].at[stripe])

    return scatter_kernel


def _t1_body(cnt_ref, x_ref, w_ref, dinv_ref, s1_ref):
    flat = jnp.sum(cnt_ref[...], axis=0)
    deg = flat[:_N] + 1.0
    dinv = lax.rsqrt(deg)[:, None]
    dinv_ref[...] = dinv
    mm = jnp.dot(x_ref[...], w_ref[...], preferred_element_type=jnp.float32)
    s1_ref[...] = dinv * mm


def _psum(p_ref):
    top = p_ref[0, 0, :_HALF, :] + p_ref[1, 1, :_HALF, :]
    bot = p_ref[1, 0, :_HALF, :] + p_ref[0, 1, :_HALF, :]
    return jnp.concatenate([top, bot], axis=0)


def _tmid_body(p_ref, sprev_ref, dinv_ref, b_ref, w_ref, snext_ref):
    dinv = dinv_ref[...]
    accv = _psum(p_ref) + sprev_ref[...]
    a = jnp.maximum(dinv * accv + b_ref[...], 0.0)
    snext_ref[...] = dinv * jnp.dot(a, w_ref[...],
                                    preferred_element_type=jnp.float32)


def _t4_body(p_ref, sprev_ref, dinv_ref, b_ref, batch_ref, wfc_ref, bfc_ref,
             out_ref):
    dinv = dinv_ref[...]
    accv = _psum(p_ref) + sprev_ref[...]
    a = jnp.maximum(dinv * accv + b_ref[...], 0.0)
    gid = lax.broadcasted_iota(jnp.int32, (_G, _N), 0)
    onehot = (batch_ref[...] == gid).astype(jnp.float32)
    sums = jnp.dot(onehot, a, preferred_element_type=jnp.float32)
    counts = jnp.sum(onehot, axis=1)[:, None]
    pooled = sums / jnp.maximum(counts, 1.0)
    out_ref[...] = jnp.dot(pooled, wfc_ref[...],
                           preferred_element_type=jnp.float32) + bfc_ref[...]


def kernel(x, edge_index, batch, W1, b1, W2, b2, W3, b3, Wfc, bfc):
    e = edge_index.shape[1]
    nchunk_w = -(-e // (_NW * _CHUNK))
    nchunk_w = -(-nchunk_w // 16) * 16
    e_pad = _NW * nchunk_w * _CHUNK
    pad = e_pad - e

    src_pad = jnp.concatenate([edge_index[0],
                               jnp.zeros((pad,), jnp.int32)])
    dst_pad = jnp.concatenate([edge_index[1],
                               jnp.full((pad,), _N, jnp.int32)])
    zeros_half = jnp.zeros((_HROWS, _D), jnp.float32)

    cnt, srcl, dstl = _make_prep_kernel(e_pad)(src_pad, dst_pad)
    nrow = e_pad // _CHUNK
    srcl = srcl.reshape(2, nrow, _CHUNK)
    dstl = dstl.reshape(2, nrow, _CHUNK)

    dinv, s1 = pl.pallas_call(
        _t1_body,
        out_shape=(jax.ShapeDtypeStruct((_N, 1), jnp.float32),
                   jax.ShapeDtypeStruct((_N, _D), jnp.float32)),
    )(cnt, x, W1)

    scatter = _make_scatter_kernel(e_pad)

    def halves(sv):
        return jnp.pad(sv.reshape(2, _HALF, _D),
                       ((0, 0), (0, _HROWS - _HALF), (0, 0)))

    def mid(s_prev, b_prev, w_next):
        p = scatter(halves(s_prev), srcl, dstl, zeros_half)
        return pl.pallas_call(
            _tmid_body,
            out_shape=jax.ShapeDtypeStruct((_N, _D), jnp.float32),
        )(p, s_prev, dinv, b_prev.reshape(1, _D), w_next)

    s2 = mid(s1, b1, W2)
    s3 = mid(s2, b2, W3)

    p3 = scatter(halves(s3), srcl, dstl, zeros_half)
    out = pl.pallas_call(
        _t4_body,
        out_shape=jax.ShapeDtypeStruct((_G, bfc.shape[0]), jnp.float32),
    )(p3, s3, dinv, b3.reshape(1, _D), batch.reshape(1, _N), Wfc,
      bfc.reshape(1, bfc.shape[0]))
    return out


# speedup vs baseline: 2.4710x; 2.4710x over previous
import functools

import jax
import jax.numpy as jnp
from jax import lax
from jax.experimental import pallas as pl
from jax.experimental.pallas import tpu as pltpu
from jax.experimental.pallas import tpu_sc as plsc

_N = 10000
_D = 128
_G = 64

_NC = 2
_NS = 16
_NW = _NC * _NS
_CHUNK = 128

_HALF = _N // 2
_HROWS = 5120
_HSTRIPE = _HROWS // _NS

_N_CNT = 10240


def _sc_mesh():
    return plsc.VectorSubcoreMesh(core_axis_name="c", subcore_axis_name="s")


def _make_prep_kernel(e_pad):
    ew = e_pad // _NW
    copies = 8
    per_copy = ew // copies
    cap = ew + 9 * 16
    assert per_copy * copies == ew and per_copy % 16 == 0 and cap % 8 == 0

    @functools.partial(
        pl.kernel,
        out_type=(
            jax.ShapeDtypeStruct((_NW, _N_CNT), jnp.float32),
            jax.ShapeDtypeStruct((4 * _NW, cap), jnp.int32),
            jax.ShapeDtypeStruct((4 * _NW, cap), jnp.int32),
            jax.ShapeDtypeStruct((_NW * 64,), jnp.int32),
        ),
        mesh=_sc_mesh(),
        scratch_types=[
            pltpu.VMEM((_N_CNT,), jnp.float32),
            pltpu.VMEM((per_copy,), jnp.int32),
            pltpu.VMEM((per_copy,), jnp.int32),
            pltpu.VMEM((cap,), jnp.int32),
            pltpu.VMEM((cap,), jnp.int32),
            pltpu.VMEM((cap,), jnp.int32),
            pltpu.VMEM((cap,), jnp.int32),
            pltpu.VMEM((cap,), jnp.int32),
            pltpu.VMEM((cap,), jnp.int32),
            pltpu.VMEM((cap,), jnp.int32),
            pltpu.VMEM((cap,), jnp.int32),
            pltpu.VMEM((64,), jnp.int32),
        ],
        compiler_params=pltpu.CompilerParams(needs_layout_passes=False),
    )
    def prep_kernel(src_hbm, dst_hbm, cnt_hbm, qsrc_hbm, qdst_hbm, qcnt_hbm,
                    cnt_v, sbuf, dbuf, sk0, sk1, sk2, sk3,
                    dk0, dk1, dk2, dk3, cstg):
        sbkt = (sk0, sk1, sk2, sk3)
        dbkt = (dk0, dk1, dk2, dk3)
        c = lax.axis_index("c")
        s = lax.axis_index("s")
        wid = s * _NC + c
        zero16 = jnp.zeros((16,), jnp.float32)
        ones16 = jnp.ones((16,), jnp.float32)
        half16 = jnp.full((16,), _HALF, jnp.int32)
        dummy16 = jnp.full((16,), _HALF, jnp.int32)

        def z_body(i, carry):
            cnt_v[pl.ds(i * 16, 16)] = zero16
            return carry
        lax.fori_loop(0, _N_CNT // 16, z_body, 0)

        def outer(j, offs):
            base = wid * ew + j * per_copy
            pltpu.sync_copy(src_hbm.at[pl.ds(base, per_copy)], sbuf)
            pltpu.sync_copy(dst_hbm.at[pl.ds(base, per_copy)], dbuf)

            def inner(k, offs2):
                sl = pl.ds(k * 16, 16)
                sv = sbuf[sl]
                dv = dbuf[sl]
                plsc.addupdate_scatter(cnt_v, (dv,), ones16)
                s_hi = sv >= half16
                d_hi = dv >= half16
                sloc = jnp.where(s_hi, sv - half16, sv)
                dloc = jnp.where(d_hi, dv - half16, dv)
                new = []
                for qh in range(4):
                    sq, dh = qh // 2, qh % 2
                    m = jnp.logical_and(s_hi == (sq == 1), d_hi == (dh == 1))
                    off = offs2[qh]
                    plsc.store_compressed(sbkt[sq * 2 + dh]
                                          .at[pl.ds(off, 16)], sloc, mask=m)
                    plsc.store_compressed(dbkt[sq * 2 + dh]
                                          .at[pl.ds(off, 16)], dloc, mask=m)
                    pc = jnp.max(plsc.all_reduce_population_count(m))
                    new.append(off + pc)
                return tuple(new)
            return lax.fori_loop(0, per_copy // 16, inner, offs)
        offs = lax.fori_loop(0, copies, outer, (0, 0, 0, 0))

        for qh in range(4):
            off = offs[qh]
            for r in range(9):
                sbkt[qh][pl.ds(off + 16 * r, 16)] = dummy16
                dbkt[qh][pl.ds(off + 16 * r, 16)] = dummy16
            nch = (off + _CHUNK - 1) // _CHUNK
            cstg[pl.ds(qh * 16, 16)] = jnp.broadcast_to(nch, (16,))
            pltpu.sync_copy(sbkt[qh], qsrc_hbm.at[qh * _NW + wid])
            pltpu.sync_copy(dbkt[qh], qdst_hbm.at[qh * _NW + wid])
        pltpu.sync_copy(cstg, qcnt_hbm.at[pl.ds(wid * 64, 64)])
        pltpu.sync_copy(cnt_v, cnt_hbm.at[wid])

    return prep_kernel


def _make_scatter_kernel(e_pad, cap):
    nch_max = cap // _CHUNK

    @functools.partial(
        pl.kernel,
        out_type=jax.ShapeDtypeStruct((_NC, 2, _HROWS, _D), jnp.float32),
        mesh=_sc_mesh(),
        scratch_types=[
            pltpu.VMEM((_CHUNK,), jnp.int32),
            pltpu.VMEM((_CHUNK,), jnp.int32),
            pltpu.VMEM((_CHUNK,), jnp.int32),
            pltpu.VMEM((_CHUNK,), jnp.int32),
            pltpu.VMEM((_CHUNK, _D), jnp.float32),
            pltpu.VMEM((_CHUNK, _D), jnp.float32),
            pltpu.VMEM((16,), jnp.int32),
            pltpu.VMEM((16,), jnp.int32),
            pltpu.VMEM_SHARED((_HROWS, _D), jnp.float32),
            pltpu.VMEM_SHARED((_HROWS, _D), jnp.float32),
            pltpu.SemaphoreType.DMA,
            pltpu.SemaphoreType.DMA,
        ],
        compiler_params=pltpu.CompilerParams(needs_layout_passes=False),
    )
    def scatter_kernel(hs_hbm, qsrc_hbm, qdst_hbm, qcnt_hbm, zeros_hbm,
                       out_hbm, sA, dA, sB, dB, rowsA, rowsB, cbA, cbB,
                       hs_sh, acc, semA, semB):
        q = lax.axis_index("c")
        s = lax.axis_index("s")
        stripe = pl.ds(s * _HSTRIPE, _HSTRIPE)

        pltpu.sync_copy(hs_hbm.at[q].at[stripe], hs_sh.at[stripe])

        for b in (0, 1):
            hb = lax.bitwise_xor(q, b)
            pltpu.sync_copy(zeros_hbm.at[stripe], acc.at[stripe])

            segs = []
            for si, (sbufs, dbufs, rows, cb, sem) in enumerate(
                    ((sA, dA, rowsA, cbA, semA), (sB, dB, rowsB, cbB, semB))):
                tseg = 2 * s + si
                row = (2 * q + hb) * _NW + tseg
                pltpu.sync_copy(
                    qcnt_hbm.at[pl.ds((tseg * 4 + 2 * q + hb) * 16, 16)], cb)
                segs.append((row, sbufs, dbufs, rows, cb, sem))
            plsc.subcore_barrier()

            nns = []
            for row, sbufs, dbufs, rows, cb, sem in segs:
                nns.append(jnp.max(cb[...]))

            def issue(seg, i):
                row, sbufs, dbufs, rows, cb, sem = seg
                pltpu.sync_copy(qsrc_hbm.at[row, pl.ds(i * _CHUNK, _CHUNK)],
                                sbufs)
                pltpu.sync_copy(qdst_hbm.at[row, pl.ds(i * _CHUNK, _CHUNK)],
                                dbufs)
                pltpu.async_copy(hs_sh.at[sbufs], rows, sem)

            def wait_scatter(seg):
                row, sbufs, dbufs, rows, cb, sem = seg
                pltpu.make_async_copy(hs_sh.at[sbufs], rows, sem).wait()
                pltpu.sync_copy(rows, acc.at[dbufs], add=True)

            for seg, nn in zip(segs, nns):
                @pl.when(nn > 0)
                def _():
                    issue(seg, 0)

            def step(i, carry):
                for seg, nn in zip(segs, nns):
                    @pl.when(i < nn)
                    def _():
                        wait_scatter(seg)

                        @pl.when(i + 1 < nn)
                        def _():
                            issue(seg, i + 1)
                return carry
            lax.fori_loop(0, jnp.maximum(nns[0], nns[1]), step, 0)
            plsc.subcore_barrier()
            pltpu.sync_copy(acc.at[stripe], out_hbm.at[q].at[b].at[stripe])

    return scatter_kernel


def _t1_body(cnt_ref, x_ref, w_ref, dinv_ref, s1_ref):
    flat = jnp.sum(cnt_ref[...], axis=0)
    deg = flat[:_N] + 1.0
    dinv = lax.rsqrt(deg)[:, None]
    dinv_ref[...] = dinv
    mm = jnp.dot(x_ref[...], w_ref[...], preferred_element_type=jnp.float32)
    s1_ref[...] = dinv * mm


def _psum(p_ref):
    top = p_ref[0, 0, :_HALF, :] + p_ref[1, 1, :_HALF, :]
    bot = p_ref[1, 0, :_HALF, :] + p_ref[0, 1, :_HALF, :]
    return jnp.concatenate([top, bot], axis=0)


def _tmid_body(p_ref, sprev_ref, dinv_ref, b_ref, w_ref, snext_ref):
    dinv = dinv_ref[...]
    accv = _psum(p_ref) + sprev_ref[...]
    a = jnp.maximum(dinv * accv + b_ref[...], 0.0)
    snext_ref[...] = dinv * jnp.dot(a, w_ref[...],
                                    preferred_element_type=jnp.float32)


def _t4_body(p_ref, sprev_ref, dinv_ref, b_ref, batch_ref, wfc_ref, bfc_ref,
             out_ref):
    dinv = dinv_ref[...]
    accv = _psum(p_ref) + sprev_ref[...]
    a = jnp.maximum(dinv * accv + b_ref[...], 0.0)
    gid = lax.broadcasted_iota(jnp.int32, (_G, _N), 0)
    onehot = (batch_ref[...] == gid).astype(jnp.float32)
    sums = jnp.dot(onehot, a, preferred_element_type=jnp.float32)
    counts = jnp.sum(onehot, axis=1)[:, None]
    pooled = sums / jnp.maximum(counts, 1.0)
    out_ref[...] = jnp.dot(pooled, wfc_ref[...],
                           preferred_element_type=jnp.float32) + bfc_ref[...]


def kernel(x, edge_index, batch, W1, b1, W2, b2, W3, b3, Wfc, bfc):
    e = edge_index.shape[1]
    nchunk_w = -(-e // (_NW * _CHUNK))
    nchunk_w = -(-nchunk_w // 16) * 16
    e_pad = _NW * nchunk_w * _CHUNK
    pad = e_pad - e
    cap = e_pad // _NW + 9 * 16

    src_pad = jnp.concatenate([edge_index[0],
                               jnp.zeros((pad,), jnp.int32)])
    dst_pad = jnp.concatenate([edge_index[1],
                               jnp.full((pad,), _N, jnp.int32)])
    zeros_half = jnp.zeros((_HROWS, _D), jnp.float32)

    cnt, qsrc, qdst, qcnt = _make_prep_kernel(e_pad)(src_pad, dst_pad)

    dinv, s1 = pl.pallas_call(
        _t1_body,
        out_shape=(jax.ShapeDtypeStruct((_N, 1), jnp.float32),
                   jax.ShapeDtypeStruct((_N, _D), jnp.float32)),
    )(cnt, x, W1)

    scatter = _make_scatter_kernel(e_pad, cap)

    def halves(sv):
        return jnp.pad(sv.reshape(2, _HALF, _D),
                       ((0, 0), (0, _HROWS - _HALF), (0, 0)))

    def mid(s_prev, b_prev, w_next):
        p = scatter(halves(s_prev), qsrc, qdst, qcnt, zeros_half)
        return pl.pallas_call(
            _tmid_body,
            out_shape=jax.ShapeDtypeStruct((_N, _D), jnp.float32),
        )(p, s_prev, dinv, b_prev.reshape(1, _D), w_next)

    s2 = mid(s1, b1, W2)
    s3 = mid(s2, b2, W3)

    p3 = scatter(halves(s3), qsrc, qdst, qcnt, zeros_half)
    out = pl.pallas_call(
        _t4_body,
        out_shape=jax.ShapeDtypeStruct((_G, bfc.shape[0]), jnp.float32),
    )(p3, s3, dinv, b3.reshape(1, _D), batch.reshape(1, _N), Wfc,
      bfc.reshape(1, bfc.shape[0]))
    return out


# speedup vs baseline: 2.7709x; 1.1214x over previous
import functools

import jax
import jax.numpy as jnp
from jax import lax
from jax.experimental import pallas as pl
from jax.experimental.pallas import tpu as pltpu
from jax.experimental.pallas import tpu_sc as plsc

_N = 10000
_D = 128
_G = 64

_NC = 2
_NS = 16
_NW = _NC * _NS
_CHUNK = 128

_HALF = _N // 2
_HROWS = 5120
_HSTRIPE = _HROWS // _NS

_N_CNT = 10240


def _sc_mesh():
    return plsc.VectorSubcoreMesh(core_axis_name="c", subcore_axis_name="s")


def _make_prep_kernel(e_pad):
    ew = e_pad // _NW
    copies = 8
    per_copy = ew // copies
    cap = ew + 9 * 16
    assert per_copy * copies == ew and per_copy % 16 == 0 and cap % 8 == 0

    @functools.partial(
        pl.kernel,
        out_type=(
            jax.ShapeDtypeStruct((_NW, _N_CNT), jnp.float32),
            jax.ShapeDtypeStruct((4 * _NW, cap), jnp.int32),
            jax.ShapeDtypeStruct((4 * _NW, cap), jnp.int32),
            jax.ShapeDtypeStruct((_NW * 64,), jnp.int32),
        ),
        mesh=_sc_mesh(),
        scratch_types=[
            pltpu.VMEM((_N_CNT,), jnp.float32),
            pltpu.VMEM((per_copy,), jnp.int32),
            pltpu.VMEM((per_copy,), jnp.int32),
            pltpu.VMEM((cap,), jnp.int32),
            pltpu.VMEM((cap,), jnp.int32),
            pltpu.VMEM((cap,), jnp.int32),
            pltpu.VMEM((cap,), jnp.int32),
            pltpu.VMEM((cap,), jnp.int32),
            pltpu.VMEM((cap,), jnp.int32),
            pltpu.VMEM((cap,), jnp.int32),
            pltpu.VMEM((cap,), jnp.int32),
            pltpu.VMEM((64,), jnp.int32),
        ],
        compiler_params=pltpu.CompilerParams(needs_layout_passes=False),
    )
    def prep_kernel(src_hbm, dst_hbm, cnt_hbm, qsrc_hbm, qdst_hbm, qcnt_hbm,
                    cnt_v, sbuf, dbuf, sk0, sk1, sk2, sk3,
                    dk0, dk1, dk2, dk3, cstg):
        sbkt = (sk0, sk1, sk2, sk3)
        dbkt = (dk0, dk1, dk2, dk3)
        c = lax.axis_index("c")
        s = lax.axis_index("s")
        wid = s * _NC + c
        zero16 = jnp.zeros((16,), jnp.float32)
        ones16 = jnp.ones((16,), jnp.float32)
        half16 = jnp.full((16,), _HALF, jnp.int32)
        dummy16 = jnp.full((16,), _HALF, jnp.int32)

        def z_body(i, carry):
            cnt_v[pl.ds(i * 16, 16)] = zero16
            return carry
        lax.fori_loop(0, _N_CNT // 16, z_body, 0)

        def outer(j, offs):
            base = wid * ew + j * per_copy
            pltpu.sync_copy(src_hbm.at[pl.ds(base, per_copy)], sbuf)
            pltpu.sync_copy(dst_hbm.at[pl.ds(base, per_copy)], dbuf)

            def inner(k, offs2):
                sl = pl.ds(k * 16, 16)
                sv = sbuf[sl]
                dv = dbuf[sl]
                plsc.addupdate_scatter(cnt_v, (dv,), ones16)
                s_hi = sv >= half16
                d_hi = dv >= half16
                sloc = jnp.where(s_hi, sv - half16, sv)
                dloc = jnp.where(d_hi, dv - half16, dv)
                new = []
                for qh in range(4):
                    sq, dh = qh // 2, qh % 2
                    m = jnp.logical_and(s_hi == (sq == 1), d_hi == (dh == 1))
                    off = offs2[qh]
                    plsc.store_compressed(sbkt[sq * 2 + dh]
                                          .at[pl.ds(off, 16)], sloc, mask=m)
                    plsc.store_compressed(dbkt[sq * 2 + dh]
                                          .at[pl.ds(off, 16)], dloc, mask=m)
                    pc = jnp.max(plsc.all_reduce_population_count(m))
                    new.append(off + pc)
                return tuple(new)
            return lax.fori_loop(0, per_copy // 16, inner, offs)
        offs = lax.fori_loop(0, copies, outer, (0, 0, 0, 0))

        for qh in range(4):
            off = offs[qh]
            for r in range(9):
                sbkt[qh][pl.ds(off + 16 * r, 16)] = dummy16
                dbkt[qh][pl.ds(off + 16 * r, 16)] = dummy16
            nch = (off + _CHUNK - 1) // _CHUNK
            cstg[pl.ds(qh * 16, 16)] = jnp.broadcast_to(nch, (16,))
            pltpu.sync_copy(sbkt[qh], qsrc_hbm.at[qh * _NW + wid])
            pltpu.sync_copy(dbkt[qh], qdst_hbm.at[qh * _NW + wid])
        pltpu.sync_copy(cstg, qcnt_hbm.at[pl.ds(wid * 64, 64)])
        pltpu.sync_copy(cnt_v, cnt_hbm.at[wid])

    return prep_kernel


def _make_scatter_kernel(e_pad, cap):
    nch_max = cap // _CHUNK

    @functools.partial(
        pl.kernel,
        out_type=jax.ShapeDtypeStruct((_NC, 2, _HROWS, _D), jnp.float32),
        mesh=_sc_mesh(),
        scratch_types=[
            pltpu.VMEM((2, _CHUNK), jnp.int32),
            pltpu.VMEM((2, _CHUNK), jnp.int32),
            pltpu.VMEM((2, _CHUNK), jnp.int32),
            pltpu.VMEM((2, _CHUNK), jnp.int32),
            pltpu.VMEM((_CHUNK, _D), jnp.float32),
            pltpu.VMEM((_CHUNK, _D), jnp.float32),
            pltpu.VMEM((16,), jnp.int32),
            pltpu.VMEM((16,), jnp.int32),
            pltpu.VMEM_SHARED((_HROWS, _D), jnp.float32),
            pltpu.VMEM_SHARED((_HROWS, _D), jnp.float32),
            pltpu.SemaphoreType.DMA,
            pltpu.SemaphoreType.DMA,
            pltpu.SemaphoreType.DMA,
            pltpu.SemaphoreType.DMA,
            pltpu.SemaphoreType.DMA,
            pltpu.SemaphoreType.DMA,
        ],
        compiler_params=pltpu.CompilerParams(needs_layout_passes=False),
    )
    def scatter_kernel(hs_hbm, qsrc_hbm, qdst_hbm, qcnt_hbm, zeros_hbm,
                       out_hbm, sA, dA, sB, dB, rowsA, rowsB, cbA, cbB,
                       hs_sh, acc, semA, semB, siA0, siA1, siB0, siB1):
        q = lax.axis_index("c")
        s = lax.axis_index("s")
        stripe = pl.ds(s * _HSTRIPE, _HSTRIPE)

        pltpu.sync_copy(hs_hbm.at[q].at[stripe], hs_sh.at[stripe])

        for b in (0, 1):
            hb = lax.bitwise_xor(q, b)
            pltpu.sync_copy(zeros_hbm.at[stripe], acc.at[stripe])

            segs = []
            for si, (sbufs, dbufs, rows, cb, sem, isems) in enumerate(
                    ((sA, dA, rowsA, cbA, semA, (siA0, siA1)),
                     (sB, dB, rowsB, cbB, semB, (siB0, siB1)))):
                tseg = 2 * s + si
                row = (2 * q + hb) * _NW + tseg
                pltpu.sync_copy(
                    qcnt_hbm.at[pl.ds((tseg * 4 + 2 * q + hb) * 16, 16)], cb)
                segs.append((row, sbufs, dbufs, rows, cb, sem, isems))
            plsc.subcore_barrier()

            nns = [jnp.max(seg[4][...]) for seg in segs]

            def issue_idx(seg, i, k):
                row, sbufs, dbufs, rows, cb, sem, isems = seg
                pltpu.async_copy(qsrc_hbm.at[row, pl.ds(i * _CHUNK, _CHUNK)],
                                 sbufs.at[k], isems[k])
                pltpu.async_copy(qdst_hbm.at[row, pl.ds(i * _CHUNK, _CHUNK)],
                                 dbufs.at[k], isems[k])

            def run_chunk(seg, i, k, nn):
                row, sbufs, dbufs, rows, cb, sem, isems = seg

                @pl.when(i + 1 < nn)
                def _():
                    issue_idx(seg, i + 1, 1 - k)
                pltpu.make_async_copy(
                    qsrc_hbm.at[row, pl.ds(i * _CHUNK, _CHUNK)],
                    sbufs.at[k], isems[k]).wait()
                pltpu.make_async_copy(
                    qdst_hbm.at[row, pl.ds(i * _CHUNK, _CHUNK)],
                    dbufs.at[k], isems[k]).wait()
                pltpu.async_copy(hs_sh.at[sbufs.at[k]], rows, sem)
                pltpu.make_async_copy(hs_sh.at[sbufs.at[k]], rows, sem).wait()
                pltpu.sync_copy(rows, acc.at[dbufs.at[k]], add=True)

            for seg, nn in zip(segs, nns):
                @pl.when(nn > 0)
                def _():
                    issue_idx(seg, 0, 0)

            def step(g, carry):
                for k in (0, 1):
                    i = 2 * g + k
                    for seg, nn in zip(segs, nns):
                        @pl.when(i < nn)
                        def _():
                            run_chunk(seg, i, k, nn)
                return carry
            maxn = jnp.maximum(nns[0], nns[1])
            lax.fori_loop(0, (maxn + 1) // 2, step, 0)
            plsc.subcore_barrier()
            pltpu.sync_copy(acc.at[stripe], out_hbm.at[q].at[b].at[stripe])

    return scatter_kernel


def _t1_body(cnt_ref, x_ref, w_ref, dinv_ref, s1_ref):
    flat = jnp.sum(cnt_ref[...], axis=0)
    deg = flat[:_N] + 1.0
    dinv = lax.rsqrt(deg)[:, None]
    dinv_ref[...] = dinv
    mm = jnp.dot(x_ref[...], w_ref[...], preferred_element_type=jnp.float32)
    s1_ref[...] = dinv * mm


def _psum(p_ref):
    top = p_ref[0, 0, :_HALF, :] + p_ref[1, 1, :_HALF, :]
    bot = p_ref[1, 0, :_HALF, :] + p_ref[0, 1, :_HALF, :]
    return jnp.concatenate([top, bot], axis=0)


def _tmid_body(p_ref, sprev_ref, dinv_ref, b_ref, w_ref, snext_ref):
    dinv = dinv_ref[...]
    accv = _psum(p_ref) + sprev_ref[...]
    a = jnp.maximum(dinv * accv + b_ref[...], 0.0)
    snext_ref[...] = dinv * jnp.dot(a, w_ref[...],
                                    preferred_element_type=jnp.float32)


def _t4_body(p_ref, sprev_ref, dinv_ref, b_ref, batch_ref, wfc_ref, bfc_ref,
             out_ref):
    dinv = dinv_ref[...]
    accv = _psum(p_ref) + sprev_ref[...]
    a = jnp.maximum(dinv * accv + b_ref[...], 0.0)
    gid = lax.broadcasted_iota(jnp.int32, (_G, _N), 0)
    onehot = (batch_ref[...] == gid).astype(jnp.float32)
    sums = jnp.dot(onehot, a, preferred_element_type=jnp.float32)
    counts = jnp.sum(onehot, axis=1)[:, None]
    pooled = sums / jnp.maximum(counts, 1.0)
    out_ref[...] = jnp.dot(pooled, wfc_ref[...],
                           preferred_element_type=jnp.float32) + bfc_ref[...]


def kernel(x, edge_index, batch, W1, b1, W2, b2, W3, b3, Wfc, bfc):
    e = edge_index.shape[1]
    nchunk_w = -(-e // (_NW * _CHUNK))
    nchunk_w = -(-nchunk_w // 16) * 16
    e_pad = _NW * nchunk_w * _CHUNK
    pad = e_pad - e
    cap = e_pad // _NW + 9 * 16

    src_pad = jnp.concatenate([edge_index[0],
                               jnp.zeros((pad,), jnp.int32)])
    dst_pad = jnp.concatenate([edge_index[1],
                               jnp.full((pad,), _N, jnp.int32)])
    zeros_half = jnp.zeros((_HROWS, _D), jnp.float32)

    cnt, qsrc, qdst, qcnt = _make_prep_kernel(e_pad)(src_pad, dst_pad)

    dinv, s1 = pl.pallas_call(
        _t1_body,
        out_shape=(jax.ShapeDtypeStruct((_N, 1), jnp.float32),
                   jax.ShapeDtypeStruct((_N, _D), jnp.float32)),
    )(cnt, x, W1)

    scatter = _make_scatter_kernel(e_pad, cap)

    def halves(sv):
        return jnp.pad(sv.reshape(2, _HALF, _D),
                       ((0, 0), (0, _HROWS - _HALF), (0, 0)))

    def mid(s_prev, b_prev, w_next):
        p = scatter(halves(s_prev), qsrc, qdst, qcnt, zeros_half)
        return pl.pallas_call(
            _tmid_body,
            out_shape=jax.ShapeDtypeStruct((_N, _D), jnp.float32),
        )(p, s_prev, dinv, b_prev.reshape(1, _D), w_next)

    s2 = mid(s1, b1, W2)
    s3 = mid(s2, b2, W3)

    p3 = scatter(halves(s3), qsrc, qdst, qcnt, zeros_half)
    out = pl.pallas_call(
        _t4_body,
        out_shape=jax.ShapeDtypeStruct((_G, bfc.shape[0]), jnp.float32),
    )(p3, s3, dinv, b3.reshape(1, _D), batch.reshape(1, _N), Wfc,
      bfc.reshape(1, bfc.shape[0]))
    return out
